# baseline (speedup 1.0000x reference)
import functools

import jax
import jax.numpy as jnp
from jax import lax
from jax.experimental import pallas as pl
from jax.experimental.pallas import tpu as pltpu
from jax.experimental.pallas import tpu_sc as plsc

_info = plsc.get_sparse_core_info()
_NC = _info.num_cores
_NS = _info.num_subcores
_NW = _NC * _NS

_CHUNK = 128


@functools.cache
def _build_gather(tot, d):
    assert tot % (_NW * _CHUNK) == 0
    n_chunks = (tot // _NW) // _CHUNK
    b_per_w = n_chunks * _CHUNK

    mesh = plsc.VectorSubcoreMesh(core_axis_name="c", subcore_axis_name="s")

    @functools.partial(
        pl.kernel,
        out_type=jax.ShapeDtypeStruct((tot, d), jnp.float32),
        mesh=mesh,
        compiler_params=pltpu.CompilerParams(use_tc_tiling_on_sc=True),
        scratch_types=[
            pltpu.VMEM((n_chunks, _CHUNK), jnp.int32),
            pltpu.VMEM((b_per_w, d), jnp.float32),
            pltpu.SemaphoreType.DMA,
        ],
    )
    def emb_kernel(idx_hbm, table_hbm, out_hbm, idx_v, rows_v, sem):
        wid = lax.axis_index("s") * _NC + lax.axis_index("c")
        pltpu.sync_copy(idx_hbm.at[wid], idx_v)
        copies = []
        for j in range(n_chunks):
            copies.append(
                pltpu.async_copy(
                    table_hbm.at[idx_v.at[j]],
                    rows_v.at[pl.ds(j * _CHUNK, _CHUNK)],
                    sem,
                )
            )
        for cp in copies:
            cp.wait()
        pltpu.sync_copy(rows_v, out_hbm.at[pl.ds(wid * b_per_w, b_per_w)])

    return emb_kernel


def kernel(x, W):
    b, p = x.shape
    d = W.shape[1]
    tot = b * p
    idx = x.reshape(_NW, (tot // _NW) // _CHUNK, _CHUNK).astype(jnp.int32)
    out = _build_gather(tot, d)(idx, W)
    return out.reshape(b, p, d)


# speedup vs baseline: 1.3048x; 1.3048x over previous
import functools

import jax
import jax.numpy as jnp
from jax import lax
from jax.experimental import pallas as pl
from jax.experimental.pallas import tpu as pltpu
from jax.experimental.pallas import tpu_sc as plsc

_info = plsc.get_sparse_core_info()
_NC = _info.num_cores
_NS = _info.num_subcores
_NW = _NC * _NS

_CHUNK = 128


@functools.cache
def _build_gather(b, p, d):
    tot = b * p
    assert tot % (_NW * _CHUNK) == 0 and (tot // _NW) % p == 0
    n_chunks = (tot // _NW) // _CHUNK
    b_per_w = n_chunks * _CHUNK
    rows_per_w = b_per_w // p

    mesh = plsc.VectorSubcoreMesh(core_axis_name="c", subcore_axis_name="s")

    @functools.partial(
        pl.kernel,
        out_type=jax.ShapeDtypeStruct((b, p, d), jnp.float32),
        mesh=mesh,
        scratch_types=[
            pltpu.VMEM((n_chunks, _CHUNK), jnp.int32),
            pltpu.VMEM((b_per_w, d), jnp.float32),
            pltpu.SemaphoreType.DMA,
            pltpu.SemaphoreType.DMA,
        ],
    )
    def emb_kernel(idx_hbm, table_hbm, out_hbm, idx_v, rows_v, sem_g, sem_s):
        wid = lax.axis_index("s") * _NC + lax.axis_index("c")
        pltpu.sync_copy(idx_hbm.at[wid], idx_v)
        gathers = []
        for j in range(n_chunks):
            gathers.append(
                pltpu.async_copy(
                    table_hbm.at[idx_v.at[j]],
                    rows_v.at[pl.ds(j * _CHUNK, _CHUNK)],
                    sem_g,
                )
            )
        for cp in gathers:
            cp.wait()
        scatters = []
        for r in range(rows_per_w):
            scatters.append(
                pltpu.async_copy(
                    rows_v.at[pl.ds(r * p, p)],
                    out_hbm.at[wid * rows_per_w + r],
                    sem_s,
                )
            )
        for cp in scatters:
            cp.wait()

    return emb_kernel


def kernel(x, W):
    b, p = x.shape
    d = W.shape[1]
    n_chunks = (b * p // _NW) // _CHUNK
    idx = x.reshape(_NW, n_chunks, _CHUNK).astype(jnp.int32)
    return _build_gather(b, p, d)(idx, W)


# speedup vs baseline: 1.7757x; 1.3609x over previous
import functools

import jax
import jax.numpy as jnp
from jax import lax
from jax.experimental import pallas as pl
from jax.experimental.pallas import tpu as pltpu
from jax.experimental.pallas import tpu_sc as plsc

_info = plsc.get_sparse_core_info()
_NC = _info.num_cores
_NS = _info.num_subcores
_NW = _NC * _NS

_CHUNK = 128


@functools.cache
def _build_gather(tot, d):
    assert tot % (_NW * _CHUNK) == 0
    n_chunks = (tot // _NW) // _CHUNK
    b_per_w = n_chunks * _CHUNK

    mesh = plsc.VectorSubcoreMesh(core_axis_name="c", subcore_axis_name="s")

    @functools.partial(
        pl.kernel,
        out_type=jax.ShapeDtypeStruct((tot, d), jnp.float32),
        mesh=mesh,
        scratch_types=[
            pltpu.VMEM((n_chunks, _CHUNK), jnp.int32),
            pltpu.VMEM((b_per_w, d), jnp.float32),
            pltpu.SemaphoreType.DMA,
        ],
    )
    def emb_kernel(idx_hbm, table_hbm, out_hbm, idx_v, rows_v, sem):
        wid = lax.axis_index("s") * _NC + lax.axis_index("c")
        pltpu.sync_copy(idx_hbm.at[wid], idx_v)
        gathers = []
        for j in range(n_chunks):
            gathers.append(
                pltpu.async_copy(
                    table_hbm.at[idx_v.at[j]],
                    rows_v.at[pl.ds(j * _CHUNK, _CHUNK)],
                    sem,
                )
            )
        for cp in gathers:
            cp.wait()
        pltpu.sync_copy(rows_v, out_hbm.at[pl.ds(wid * b_per_w, b_per_w)])

    return emb_kernel


def kernel(x, W):
    b, p = x.shape
    d = W.shape[1]
    tot = b * p
    idx = x.T.reshape(_NW, (tot // _NW) // _CHUNK, _CHUNK).astype(jnp.int32)
    out_t = _build_gather(tot, d)(idx, W)
    return out_t.reshape(p, b, d).transpose(1, 0, 2)
